# baseline (speedup 1.0000x reference)
import functools

import jax
import jax.numpy as jnp
from jax import lax
from jax.experimental import pallas as pl
from jax.experimental.pallas import tpu as pltpu
from jax.experimental.pallas import tpu_sc as plsc

_BATCH = 16384
_SEQ = 200
_DIM = 64
_LANES = 16
_TOTAL = _BATCH * _SEQ
_G = 128
_NGROUPS = _TOTAL // _G
_NC = 2
_NS = 16
_NW = _NC * _NS
_PER_W = _NGROUPS // _NW
_CHUNK = _G
_NCHUNK = _PER_W
_IDSB = 16
_NBUF = 8
_VOCAB = 100


@functools.lru_cache(maxsize=1)
def _make_sc_lookup():
    mesh = plsc.VectorSubcoreMesh(core_axis_name="c", subcore_axis_name="s")

    @functools.partial(
        pl.kernel,
        mesh=mesh,
        compiler_params=pltpu.CompilerParams(use_tc_tiling_on_sc=False),
        out_type=jax.ShapeDtypeStruct((_TOTAL, _DIM), jnp.float32),
        scratch_types=[
            pltpu.VMEM((_VOCAB, _DIM), jnp.float32),
            pltpu.VMEM((2 * _IDSB, _G), jnp.int32),
            pltpu.VMEM((_NBUF, _CHUNK, _DIM), jnp.float32),
            pltpu.SemaphoreType.DMA((_NBUF,)),
            pltpu.SemaphoreType.DMA((2,)),
        ],
    )
    def lookup(ids_hbm, table_hbm, out_hbm, table_v, ids_v, rows_v, sem,
               isem):
        sems = [sem.at[b] for b in range(_NBUF)]
        wid = lax.axis_index("s") * _NC + lax.axis_index("c")
        base_g = wid * _PER_W

        pltpu.sync_copy(table_hbm, table_v)

        def ids_op(block, slot):
            return pltpu.make_async_copy(
                ids_hbm.at[pl.ds(base_g + block * _IDSB, _IDSB)],
                ids_v.at[pl.ds(slot * _IDSB, _IDSB)],
                isem.at[slot],
            )

        def store_op(c, buf, sem):
            return pltpu.make_async_copy(
                rows_v.at[buf],
                out_hbm.at[pl.ds((base_g + c) * _G, _CHUNK)],
                sem,
            )

        def gather_rows(c, buf):
            grow = lax.rem(c, 2 * _IDSB)

            @plsc.parallel_loop(0, _G // _LANES, unroll=4)
            def _(i):
                idvec = ids_v[grow, pl.ds(i * _LANES, _LANES)]
                for u in range(_LANES):
                    rid = idvec[u]
                    r = i * _LANES + u
                    for j in range(_DIM // _LANES):
                        rows_v[buf, r, pl.ds(j * _LANES, _LANES)] = (
                            table_v[rid, pl.ds(j * _LANES, _LANES)]
                        )

        ids_op(0, 0).start()

        def body(c, carry):
            buf = lax.rem(c, _NBUF)
            for b in range(_NBUF):

                @pl.when(jnp.logical_and(buf == b, c >= _NBUF))
                def _(b=b):
                    store_op(c - _NBUF, b, sems[b]).wait()

            @pl.when(lax.rem(c, _IDSB) == 0)
            def _():
                blk = c // _IDSB
                for s in range(2):

                    @pl.when(lax.rem(blk, 2) == s)
                    def _(s=s):
                        ids_op(blk, s).wait()

                        @pl.when(c + _IDSB < _NCHUNK)
                        def _(s=s):
                            ids_op(blk + 1, 1 - s).start()

            gather_rows(c, buf)

            for b in range(_NBUF):

                @pl.when(buf == b)
                def _(b=b):
                    store_op(c, b, sems[b]).start()

            return carry

        lax.fori_loop(0, _NCHUNK, body, 0)

        for b in range(_NBUF):
            c_last = _NCHUNK - _NBUF + b
            buf = c_last % _NBUF
            store_op(c_last, buf, sems[buf]).wait()

    return lookup


def kernel(segmentation_ids, table):
    ids2d = segmentation_ids.reshape(_NGROUPS, _G).astype(jnp.int32)
    out = _make_sc_lookup()(ids2d, table)
    return out.reshape(_BATCH, _SEQ, _DIM)


# speedup vs baseline: 1.1526x; 1.1526x over previous
import functools

import jax
import jax.numpy as jnp
from jax import lax
from jax.experimental import pallas as pl
from jax.experimental.pallas import tpu as pltpu
from jax.experimental.pallas import tpu_sc as plsc

_BATCH = 16384
_SEQ = 200
_DIM = 64
_LANES = 16
_TOTAL = _BATCH * _SEQ
_G = 128
_NGROUPS = _TOTAL // _G
_NC = 2
_NS = 16
_NW = _NC * _NS
_PER_W = _NGROUPS // _NW
_CHUNK = _G
_NCHUNK = _PER_W
_IDSB = 16
_NBUF = 6
_VOCAB = 100


@functools.lru_cache(maxsize=1)
def _make_sc_lookup():
    mesh = plsc.VectorSubcoreMesh(core_axis_name="c", subcore_axis_name="s")

    @functools.partial(
        pl.kernel,
        mesh=mesh,
        compiler_params=pltpu.CompilerParams(use_tc_tiling_on_sc=False),
        out_type=jax.ShapeDtypeStruct((_TOTAL, _DIM), jnp.float32),
        scratch_types=[
            pltpu.VMEM((_VOCAB, _DIM), jnp.float32),
            pltpu.VMEM((2 * _IDSB, _G), jnp.int32),
            pltpu.VMEM((_NBUF, _CHUNK, _DIM), jnp.float32),
            pltpu.SemaphoreType.DMA((_NBUF,)),
            pltpu.SemaphoreType.DMA((2,)),
        ],
    )
    def lookup(ids_hbm, table_hbm, out_hbm, table_v, ids_v, rows_v, sem,
               isem):
        sems = [sem.at[b] for b in range(_NBUF)]
        wid = lax.axis_index("s") * _NC + lax.axis_index("c")
        base_g = wid * _PER_W

        pltpu.sync_copy(table_hbm, table_v)

        def ids_op(block, slot):
            return pltpu.make_async_copy(
                ids_hbm.at[pl.ds(base_g + block * _IDSB, _IDSB)],
                ids_v.at[pl.ds(slot * _IDSB, _IDSB)],
                isem.at[slot],
            )

        def store_op(c, buf, sem):
            return pltpu.make_async_copy(
                rows_v.at[buf],
                out_hbm.at[pl.ds((base_g + c) * _G, _CHUNK)],
                sem,
            )

        def gather_rows(c, buf):
            grow = lax.rem(c, 2 * _IDSB)

            @plsc.parallel_loop(0, _G // _LANES, unroll=2)
            def _(i):
                idvec = ids_v[grow, pl.ds(i * _LANES, _LANES)]
                for u in range(_LANES):
                    r = i * _LANES + u
                    rid = idvec[u]
                    for j in range(_DIM // _LANES):
                        rows_v[buf, r, pl.ds(j * _LANES, _LANES)] = (
                            table_v[rid, pl.ds(j * _LANES, _LANES)]
                        )

        ids_op(0, 0).start()

        def body(c, carry):
            buf = lax.rem(c, _NBUF)
            for b in range(_NBUF):

                @pl.when(jnp.logical_and(buf == b, c >= _NBUF))
                def _(b=b):
                    store_op(c - _NBUF, b, sems[b]).wait()

            @pl.when(lax.rem(c, _IDSB) == 0)
            def _():
                blk = c // _IDSB
                for s in range(2):

                    @pl.when(lax.rem(blk, 2) == s)
                    def _(s=s):
                        ids_op(blk, s).wait()

                        @pl.when(c + _IDSB < _NCHUNK)
                        def _(s=s):
                            ids_op(blk + 1, 1 - s).start()

            gather_rows(c, buf)

            for b in range(_NBUF):

                @pl.when(buf == b)
                def _(b=b):
                    store_op(c, b, sems[b]).start()

            return carry

        lax.fori_loop(0, _NCHUNK, body, 0)

        for b in range(_NBUF):
            c_last = _NCHUNK - _NBUF + b
            buf = c_last % _NBUF
            store_op(c_last, buf, sems[buf]).wait()

    return lookup


def kernel(segmentation_ids, table):
    ids2d = segmentation_ids.reshape(_NGROUPS, _G).astype(jnp.int32)
    out = _make_sc_lookup()(ids2d, table)
    return out.reshape(_BATCH, _SEQ, _DIM)
